# baseline (speedup 1.0000x reference)
import dataclasses
import functools

import jax
import jax.numpy as jnp
from jax import lax
from jax.experimental import pallas as pl
from jax.experimental.pallas import tpu as pltpu
from jax.experimental.pallas import tpu_sc as plsc

_N = 10000
_E = 320000
_D = 128
_NC = 2
_NS = 16
_NW = _NC * _NS
_EPW = _E // _NW
_B = 80
_C = _EPW // _B
_RQ = 624
_RL = _N - (_NS - 1) * _RQ


_mesh = plsc.VectorSubcoreMesh(core_axis_name="c", subcore_axis_name="s")

_cp = pltpu.CompilerParams()
if "needs_layout_passes" in pltpu.CompilerParams.__dataclass_fields__:
    _cp = dataclasses.replace(_cp, needs_layout_passes=False)


_NBUF = 2


@functools.partial(
    pl.kernel,
    out_type=jax.ShapeDtypeStruct((_NC, _N, _D), jnp.float32),
    mesh=_mesh,
    compiler_params=_cp,
    scratch_types=[
        pltpu.VMEM((_EPW,), jnp.int32),
        pltpu.VMEM((_EPW,), jnp.float32),
        pltpu.VMEM((_NBUF, _B), jnp.int32),
        pltpu.VMEM((_NBUF, _B), jnp.int32),
        pltpu.VMEM((_NBUF, _B, _D), jnp.float32),
        pltpu.VMEM_SHARED((_N, _D), jnp.float32),
    ]
    + [pltpu.SemaphoreType.DMA] * _NBUF,
)
def _sc_edge_pass(g_hbm, sd_hbm, w_hbm, zero_hbm, out_hbm,
                  sd_v, w_v, srcr_v, dstr_v, rows_v, acc_s, *sems):
    cid = lax.axis_index("c")
    sid = lax.axis_index("s")
    wid = sid * _NC + cid
    gsem = sems

    row0 = sid * _RQ

    @pl.when(sid < _NS - 1)
    def _():
        pltpu.sync_copy(zero_hbm.at[pl.ds(row0, _RQ)],
                        acc_s.at[pl.ds(row0, _RQ)])

    @pl.when(sid == _NS - 1)
    def _():
        pltpu.sync_copy(zero_hbm.at[pl.ds(row0, _RL)],
                        acc_s.at[pl.ds(row0, _RL)])

    pltpu.sync_copy(sd_hbm.at[wid], sd_v)
    pltpu.sync_copy(w_hbm.at[wid], w_v)

    plsc.subcore_barrier()

    def unpack_src(j, b):
        for u in range(_B // 16):
            p = sd_v[pl.ds(j * _B + u * 16, 16)]
            srcr_v[b, pl.ds(u * 16, 16)] = p & jnp.int32(0xFFFF)

    def unpack_dst(j, b):
        for u in range(_B // 16):
            p = sd_v[pl.ds(j * _B + u * 16, 16)]
            dstr_v[b, pl.ds(u * 16, 16)] = lax.shift_right_logical(
                p, jnp.int32(16))

    def start_gather(j, b):
        pltpu.async_copy(g_hbm.at[srcr_v.at[b]], rows_v.at[b], gsem[b])

    def wait_gather(j, b):
        pltpu.make_async_copy(g_hbm.at[srcr_v.at[b]], rows_v.at[b],
                              gsem[b]).wait()

    def do_scatter(j, b):
        pltpu.sync_copy(rows_v.at[b], acc_s.at[dstr_v.at[b]], add=True)

    def mul(j, b):
        @pl.loop(0, _B)
        def _(i):
            wsp = plsc.load_gather(
                w_v, [jnp.full((16,), j * _B + i, jnp.int32)])
            for c in range(_D // 16):
                sl = pl.ds(c * 16, 16)
                rows_v[b, i, sl] = rows_v[b, i, sl] * wsp

    def body(j, b, last):
        unpack_src(j, b)
        pltpu.sync_copy(g_hbm.at[srcr_v.at[b]], rows_v.at[b])
        mul(j, b)
        unpack_dst(j, b)
        do_scatter(j, b)

    @pl.loop(0, _C)
    def _(j):
        body(j, 0, last=False)

    plsc.subcore_barrier()

    @pl.when(sid < _NS - 1)
    def _():
        pltpu.sync_copy(acc_s.at[pl.ds(row0, _RQ)],
                        out_hbm.at[cid, pl.ds(row0, _RQ)])

    @pl.when(sid == _NS - 1)
    def _():
        pltpu.sync_copy(acc_s.at[pl.ds(row0, _RL)],
                        out_hbm.at[cid, pl.ds(row0, _RL)])


def _mm_first_block(h_ref, a_ref, w_ref, o_ref):
    o_ref[...] = a_ref[...] * jnp.dot(h_ref[...], w_ref[...],
                                      preferred_element_type=jnp.float32)


def _mm_mid_block(p_ref, g_ref, a_ref, b_ref, bias_ref, w_ref, o_ref):
    t = b_ref[...] * (p_ref[0] + p_ref[1] + g_ref[...]) + bias_ref[...]
    o_ref[...] = a_ref[...] * jnp.dot(t, w_ref[...],
                                      preferred_element_type=jnp.float32)


def _final_block(p_ref, g_ref, b_ref, bias_ref, o_ref):
    o_ref[...] = (b_ref[...] * (p_ref[0] + p_ref[1] + g_ref[...])
                  + bias_ref[...])


_f32 = jnp.float32
_out_nd = jax.ShapeDtypeStruct((_N, _D), _f32)


def kernel(inputs, edge_index, edge_weight, Ws, bs):
    src = edge_index[0]
    dst = edge_index[1]
    w = jnp.where(src == dst, jnp.zeros_like(edge_weight), edge_weight)
    deg_out = jax.ops.segment_sum(w, src, num_segments=_N) + 1.0
    deg_in = jax.ops.segment_sum(w, dst, num_segments=_N) + 1.0
    a = lax.rsqrt(deg_out)[:, None]
    b = lax.rsqrt(deg_in)[:, None]
    bias = bs[:, None, :]

    sd3 = (src | (dst << 16)).reshape(_NW, _EPW)
    w3 = w.reshape(_NW, _EPW)
    zeros = jnp.zeros((_N, _D), _f32)

    g = pl.pallas_call(_mm_first_block, out_shape=_out_nd)(inputs, a, Ws[0])
    for l in range(3):
        parts = _sc_edge_pass(g, sd3, w3, zeros)
        if l < 2:
            g = pl.pallas_call(_mm_mid_block, out_shape=_out_nd)(
                parts, g, a, b, bias[l], Ws[l + 1])
        else:
            h = pl.pallas_call(_final_block, out_shape=_out_nd)(
                parts, g, b, bias[l])
    return h


# speedup vs baseline: 1.3203x; 1.3203x over previous
import dataclasses
import functools

import jax
import jax.numpy as jnp
from jax import lax
from jax.experimental import pallas as pl
from jax.experimental.pallas import tpu as pltpu
from jax.experimental.pallas import tpu_sc as plsc

_N = 10000
_E = 320000
_D = 128
_NC = 2
_NS = 16
_NW = _NC * _NS
_EPW = _E // _NW
_B = 80
_C = _EPW // _B
_RQ = 624
_RL = _N - (_NS - 1) * _RQ


_mesh = plsc.VectorSubcoreMesh(core_axis_name="c", subcore_axis_name="s")

_cp = pltpu.CompilerParams()
if "needs_layout_passes" in pltpu.CompilerParams.__dataclass_fields__:
    _cp = dataclasses.replace(_cp, needs_layout_passes=False)


_NBUF = 2


@functools.partial(
    pl.kernel,
    out_type=jax.ShapeDtypeStruct((_NC, _N, _D), jnp.float32),
    mesh=_mesh,
    compiler_params=_cp,
    scratch_types=[
        pltpu.VMEM((_EPW,), jnp.int32),
        pltpu.VMEM((_EPW,), jnp.float32),
        pltpu.VMEM((_NBUF, _B), jnp.int32),
        pltpu.VMEM((_NBUF, _B), jnp.int32),
        pltpu.VMEM((_NBUF, _B, _D), jnp.float32),
        pltpu.VMEM_SHARED((_N, _D), jnp.float32),
    ]
    + [pltpu.SemaphoreType.DMA] * _NBUF,
)
def _sc_edge_pass(g_hbm, sd_hbm, w_hbm, zero_hbm, out_hbm,
                  sd_v, w_v, srcr_v, dstr_v, rows_v, acc_s, *sems):
    cid = lax.axis_index("c")
    sid = lax.axis_index("s")
    wid = sid * _NC + cid
    gsem = sems

    row0 = sid * _RQ

    @pl.when(sid < _NS - 1)
    def _():
        pltpu.sync_copy(zero_hbm.at[pl.ds(row0, _RQ)],
                        acc_s.at[pl.ds(row0, _RQ)])

    @pl.when(sid == _NS - 1)
    def _():
        pltpu.sync_copy(zero_hbm.at[pl.ds(row0, _RL)],
                        acc_s.at[pl.ds(row0, _RL)])

    pltpu.sync_copy(sd_hbm.at[wid], sd_v)
    pltpu.sync_copy(w_hbm.at[wid], w_v)

    plsc.subcore_barrier()

    def unpack_src(j, b):
        for u in range(_B // 16):
            p = sd_v[pl.ds(j * _B + u * 16, 16)]
            srcr_v[b, pl.ds(u * 16, 16)] = p & jnp.int32(0xFFFF)

    def unpack_dst(j, b):
        for u in range(_B // 16):
            p = sd_v[pl.ds(j * _B + u * 16, 16)]
            dstr_v[b, pl.ds(u * 16, 16)] = lax.shift_right_logical(
                p, jnp.int32(16))

    def start_gather(j, b):
        pltpu.async_copy(g_hbm.at[srcr_v.at[b]], rows_v.at[b], gsem[b])

    def wait_gather(j, b):
        pltpu.make_async_copy(g_hbm.at[srcr_v.at[b]], rows_v.at[b],
                              gsem[b]).wait()

    def do_scatter(j, b):
        pltpu.sync_copy(rows_v.at[b], acc_s.at[dstr_v.at[b]], add=True)

    def mul(j, b):
        @pl.loop(0, _B)
        def _(i):
            wsp = plsc.load_gather(
                w_v, [jnp.full((16,), j * _B + i, jnp.int32)])
            for c in range(_D // 16):
                sl = pl.ds(c * 16, 16)
                rows_v[b, i, sl] = rows_v[b, i, sl] * wsp

    def tail(j, b):
        mul(j, b)
        unpack_dst(j, b)
        do_scatter(j, b)

    unpack_src(0, 0)
    start_gather(0, 0)

    @pl.loop(0, _C - 1, step=_NBUF)
    def _(j0):
        for b in range(_NBUF):
            j = j0 + b
            nb = 1 - b
            unpack_src(j + 1, nb)
            start_gather(j + 1, nb)
            wait_gather(j, b)
            tail(j, b)

    wait_gather(_C - 1, (_C - 1) % _NBUF)
    tail(_C - 1, (_C - 1) % _NBUF)

    plsc.subcore_barrier()

    @pl.when(sid < _NS - 1)
    def _():
        pltpu.sync_copy(acc_s.at[pl.ds(row0, _RQ)],
                        out_hbm.at[cid, pl.ds(row0, _RQ)])

    @pl.when(sid == _NS - 1)
    def _():
        pltpu.sync_copy(acc_s.at[pl.ds(row0, _RL)],
                        out_hbm.at[cid, pl.ds(row0, _RL)])


def _mm_first_block(h_ref, a_ref, w_ref, o_ref):
    o_ref[...] = a_ref[...] * jnp.dot(h_ref[...], w_ref[...],
                                      preferred_element_type=jnp.float32)


def _mm_mid_block(p_ref, g_ref, a_ref, b_ref, bias_ref, w_ref, o_ref):
    t = b_ref[...] * (p_ref[0] + p_ref[1] + g_ref[...]) + bias_ref[...]
    o_ref[...] = a_ref[...] * jnp.dot(t, w_ref[...],
                                      preferred_element_type=jnp.float32)


def _final_block(p_ref, g_ref, b_ref, bias_ref, o_ref):
    o_ref[...] = (b_ref[...] * (p_ref[0] + p_ref[1] + g_ref[...])
                  + bias_ref[...])


_f32 = jnp.float32
_out_nd = jax.ShapeDtypeStruct((_N, _D), _f32)


def kernel(inputs, edge_index, edge_weight, Ws, bs):
    src = edge_index[0]
    dst = edge_index[1]
    w = jnp.where(src == dst, jnp.zeros_like(edge_weight), edge_weight)
    deg_out = jax.ops.segment_sum(w, src, num_segments=_N) + 1.0
    deg_in = jax.ops.segment_sum(w, dst, num_segments=_N) + 1.0
    a = lax.rsqrt(deg_out)[:, None]
    b = lax.rsqrt(deg_in)[:, None]
    bias = bs[:, None, :]

    sd3 = (src | (dst << 16)).reshape(_NW, _EPW)
    w3 = w.reshape(_NW, _EPW)
    zeros = jnp.zeros((_N, _D), _f32)

    g = pl.pallas_call(_mm_first_block, out_shape=_out_nd)(inputs, a, Ws[0])
    for l in range(3):
        parts = _sc_edge_pass(g, sd3, w3, zeros)
        if l < 2:
            g = pl.pallas_call(_mm_mid_block, out_shape=_out_nd)(
                parts, g, a, b, bias[l], Ws[l + 1])
        else:
            h = pl.pallas_call(_final_block, out_shape=_out_nd)(
                parts, g, b, bias[l])
    return h


# speedup vs baseline: 2.5179x; 1.9071x over previous
import dataclasses
import functools

import jax
import jax.numpy as jnp
from jax import lax
from jax.experimental import pallas as pl
from jax.experimental.pallas import tpu as pltpu
from jax.experimental.pallas import tpu_sc as plsc

_N = 10000
_E = 320000
_D = 128
_NC = 2
_NS = 16
_NW = _NC * _NS
_EPW = _E // _NW
_B = 80
_C = _EPW // _B
_RQ = 624
_RL = _N - (_NS - 1) * _RQ


_mesh = plsc.VectorSubcoreMesh(core_axis_name="c", subcore_axis_name="s")


_cp = pltpu.CompilerParams()
if "needs_layout_passes" in pltpu.CompilerParams.__dataclass_fields__:
    _cp = dataclasses.replace(_cp, needs_layout_passes=False)


_NBUF = 2


@functools.partial(
    pl.kernel,
    out_type=jax.ShapeDtypeStruct((_NW, 2, _N), jnp.float32),
    mesh=_mesh,
    compiler_params=_cp,
    scratch_types=[
        pltpu.VMEM((_EPW,), jnp.int32),
        pltpu.VMEM((_EPW,), jnp.float32),
        pltpu.VMEM((_N,), jnp.float32),
        pltpu.VMEM((_N,), jnp.float32),
    ],
)
def _sc_degrees(sd_hbm, w_hbm, out_hbm, sd_v, w_v, dego_v, degi_v):
    cid = lax.axis_index("c")
    sid = lax.axis_index("s")
    wid = sid * _NC + cid

    pltpu.sync_copy(sd_hbm.at[wid], sd_v)
    pltpu.sync_copy(w_hbm.at[wid], w_v)

    zero16 = jnp.zeros((16,), jnp.float32)

    @pl.loop(0, _N // 16)
    def _(i):
        dego_v[pl.ds(i * 16, 16)] = zero16
        degi_v[pl.ds(i * 16, 16)] = zero16

    @pl.loop(0, _EPW // 16)
    def _(i):
        p = sd_v[pl.ds(i * 16, 16)]
        s = p & jnp.int32(0xFFFF)
        d = lax.shift_right_logical(p, jnp.int32(16))
        wv = w_v[pl.ds(i * 16, 16)]
        plsc.addupdate_scatter(dego_v, [s], wv)
        plsc.addupdate_scatter(degi_v, [d], wv)

    pltpu.sync_copy(dego_v, out_hbm.at[wid, 0])
    pltpu.sync_copy(degi_v, out_hbm.at[wid, 1])


@functools.partial(
    pl.kernel,
    out_type=jax.ShapeDtypeStruct((_NC, _N, _D), jnp.float32),
    mesh=_mesh,
    compiler_params=_cp,
    scratch_types=[
        pltpu.VMEM((_EPW,), jnp.int32),
        pltpu.VMEM((_EPW,), jnp.float32),
        pltpu.VMEM((_NBUF, _B), jnp.int32),
        pltpu.VMEM((_NBUF, _B), jnp.int32),
        pltpu.VMEM((_NBUF, _B, _D), jnp.float32),
        pltpu.VMEM_SHARED((_N, _D), jnp.float32),
    ]
    + [pltpu.SemaphoreType.DMA] * _NBUF,
)
def _sc_edge_pass(g_hbm, sd_hbm, w_hbm, zero_hbm, out_hbm,
                  sd_v, w_v, srcr_v, dstr_v, rows_v, acc_s, *sems):
    cid = lax.axis_index("c")
    sid = lax.axis_index("s")
    wid = sid * _NC + cid
    gsem = sems

    row0 = sid * _RQ

    @pl.when(sid < _NS - 1)
    def _():
        pltpu.sync_copy(zero_hbm.at[pl.ds(row0, _RQ)],
                        acc_s.at[pl.ds(row0, _RQ)])

    @pl.when(sid == _NS - 1)
    def _():
        pltpu.sync_copy(zero_hbm.at[pl.ds(row0, _RL)],
                        acc_s.at[pl.ds(row0, _RL)])

    pltpu.sync_copy(sd_hbm.at[wid], sd_v)
    pltpu.sync_copy(w_hbm.at[wid], w_v)

    plsc.subcore_barrier()

    def unpack_src(j, b):
        for u in range(_B // 16):
            p = sd_v[pl.ds(j * _B + u * 16, 16)]
            srcr_v[b, pl.ds(u * 16, 16)] = p & jnp.int32(0xFFFF)

    def unpack_dst(j, b):
        for u in range(_B // 16):
            p = sd_v[pl.ds(j * _B + u * 16, 16)]
            dstr_v[b, pl.ds(u * 16, 16)] = lax.shift_right_logical(
                p, jnp.int32(16))

    def start_gather(j, b):
        pltpu.async_copy(g_hbm.at[srcr_v.at[b]], rows_v.at[b], gsem[b])

    def wait_gather(j, b):
        pltpu.make_async_copy(g_hbm.at[srcr_v.at[b]], rows_v.at[b],
                              gsem[b]).wait()

    def do_scatter(j, b):
        pltpu.sync_copy(rows_v.at[b], acc_s.at[dstr_v.at[b]], add=True)

    def mul(j, b):
        @pl.loop(0, _B)
        def _(i):
            wsp = plsc.load_gather(
                w_v, [jnp.full((16,), j * _B + i, jnp.int32)])
            for c in range(_D // 16):
                sl = pl.ds(c * 16, 16)
                rows_v[b, i, sl] = rows_v[b, i, sl] * wsp

    def tail(j, b):
        mul(j, b)
        unpack_dst(j, b)
        do_scatter(j, b)

    unpack_src(0, 0)
    start_gather(0, 0)

    @pl.loop(0, _C - 1, step=_NBUF)
    def _(j0):
        for b in range(_NBUF):
            j = j0 + b
            nb = 1 - b
            unpack_src(j + 1, nb)
            start_gather(j + 1, nb)
            wait_gather(j, b)
            tail(j, b)

    wait_gather(_C - 1, (_C - 1) % _NBUF)
    tail(_C - 1, (_C - 1) % _NBUF)

    plsc.subcore_barrier()

    @pl.when(sid < _NS - 1)
    def _():
        pltpu.sync_copy(acc_s.at[pl.ds(row0, _RQ)],
                        out_hbm.at[cid, pl.ds(row0, _RQ)])

    @pl.when(sid == _NS - 1)
    def _():
        pltpu.sync_copy(acc_s.at[pl.ds(row0, _RL)],
                        out_hbm.at[cid, pl.ds(row0, _RL)])


def _mm_first_block(h_ref, w_ref, o_ref):
    o_ref[...] = jnp.dot(h_ref[...], w_ref[...],
                         preferred_element_type=jnp.float32)


def _mm_mid_block(p_ref, g_ref, a_ref, b_ref, bias_ref, w_ref, o_ref):
    t = b_ref[...] * (p_ref[0] + p_ref[1] + g_ref[...]) + bias_ref[...]
    o_ref[...] = a_ref[...] * jnp.dot(t, w_ref[...],
                                      preferred_element_type=jnp.float32)


def _final_block(p_ref, g_ref, b_ref, bias_ref, o_ref):
    o_ref[...] = (b_ref[...] * (p_ref[0] + p_ref[1] + g_ref[...])
                  + bias_ref[...])


_f32 = jnp.float32
_out_nd = jax.ShapeDtypeStruct((_N, _D), _f32)


def kernel(inputs, edge_index, edge_weight, Ws, bs):
    src = edge_index[0]
    dst = edge_index[1]
    w = jnp.where(src == dst, jnp.zeros_like(edge_weight), edge_weight)
    bias = bs[:, None, :]

    sd3 = (src | (dst << 16)).reshape(_NW, _EPW)
    w3 = w.reshape(_NW, _EPW)
    zeros = jnp.zeros((_N, _D), _f32)

    deg = _sc_degrees(sd3, w3).sum(axis=0)
    a = lax.rsqrt(deg[0] + 1.0)[:, None]
    b = lax.rsqrt(deg[1] + 1.0)[:, None]

    g = a * pl.pallas_call(_mm_first_block, out_shape=_out_nd)(inputs, Ws[0])
    for l in range(3):
        parts = _sc_edge_pass(g, sd3, w3, zeros)
        if l < 2:
            g = pl.pallas_call(_mm_mid_block, out_shape=_out_nd)(
                parts, g, a, b, bias[l], Ws[l + 1])
        else:
            h = pl.pallas_call(_final_block, out_shape=_out_nd)(
                parts, g, b, bias[l])
    return h


# speedup vs baseline: 2.5500x; 1.0128x over previous
import dataclasses
import functools

import jax
import jax.numpy as jnp
from jax import lax
from jax.experimental import pallas as pl
from jax.experimental.pallas import tpu as pltpu
from jax.experimental.pallas import tpu_sc as plsc

_N = 10000
_E = 320000
_D = 128
_NC = 2
_NS = 16
_NW = _NC * _NS
_EPW = _E // _NW
_B = 80
_C = _EPW // _B
_RQ = 624
_RL = _N - (_NS - 1) * _RQ


_mesh = plsc.VectorSubcoreMesh(core_axis_name="c", subcore_axis_name="s")


_cp = pltpu.CompilerParams()
if "needs_layout_passes" in pltpu.CompilerParams.__dataclass_fields__:
    _cp = dataclasses.replace(_cp, needs_layout_passes=False)


_NBUF = 2


@functools.partial(
    pl.kernel,
    out_type=jax.ShapeDtypeStruct((_NW, 2, _N), jnp.float32),
    mesh=_mesh,
    compiler_params=_cp,
    scratch_types=[
        pltpu.VMEM((_EPW,), jnp.int32),
        pltpu.VMEM((_EPW,), jnp.float32),
        pltpu.VMEM((_N,), jnp.float32),
        pltpu.VMEM((_N,), jnp.float32),
    ],
)
def _sc_degrees(sd_hbm, w_hbm, out_hbm, sd_v, w_v, dego_v, degi_v):
    cid = lax.axis_index("c")
    sid = lax.axis_index("s")
    wid = sid * _NC + cid

    pltpu.sync_copy(sd_hbm.at[wid], sd_v)
    pltpu.sync_copy(w_hbm.at[wid], w_v)

    zero16 = jnp.zeros((16,), jnp.float32)

    @pl.loop(0, _N // 16)
    def _(i):
        dego_v[pl.ds(i * 16, 16)] = zero16
        degi_v[pl.ds(i * 16, 16)] = zero16

    @pl.loop(0, _EPW // 16)
    def _(i):
        p = sd_v[pl.ds(i * 16, 16)]
        s = p & jnp.int32(0xFFFF)
        d = lax.shift_right_logical(p, jnp.int32(16))
        wv = w_v[pl.ds(i * 16, 16)]
        plsc.addupdate_scatter(dego_v, [s], wv)
        plsc.addupdate_scatter(degi_v, [d], wv)

    pltpu.sync_copy(dego_v, out_hbm.at[wid, 0])
    pltpu.sync_copy(degi_v, out_hbm.at[wid, 1])


@functools.partial(
    pl.kernel,
    out_type=jax.ShapeDtypeStruct((_NC, _N, _D), jnp.float32),
    mesh=_mesh,
    compiler_params=_cp,
    scratch_types=[
        pltpu.VMEM((_EPW,), jnp.int32),
        pltpu.VMEM((_EPW,), jnp.float32),
        pltpu.VMEM((_NBUF, _B), jnp.int32),
        pltpu.VMEM((_NBUF, _B), jnp.int32),
        pltpu.VMEM((_NBUF, _B, _D), jnp.float32),
        pltpu.VMEM_SHARED((_N, _D), jnp.float32),
    ]
    + [pltpu.SemaphoreType.DMA] * (2 * _NBUF),
)
def _sc_edge_pass(g_hbm, sd_hbm, w_hbm, zero_hbm, out_hbm,
                  sd_v, w_v, srcr_v, dstr_v, rows_v, acc_s, *sems):
    cid = lax.axis_index("c")
    sid = lax.axis_index("s")
    wid = sid * _NC + cid
    gsem = sems[:_NBUF]
    ssem = sems[_NBUF:]

    row0 = sid * _RQ

    @pl.when(sid < _NS - 1)
    def _():
        pltpu.sync_copy(zero_hbm.at[pl.ds(row0, _RQ)],
                        acc_s.at[pl.ds(row0, _RQ)])

    @pl.when(sid == _NS - 1)
    def _():
        pltpu.sync_copy(zero_hbm.at[pl.ds(row0, _RL)],
                        acc_s.at[pl.ds(row0, _RL)])

    pltpu.sync_copy(sd_hbm.at[wid], sd_v)
    pltpu.sync_copy(w_hbm.at[wid], w_v)

    plsc.subcore_barrier()

    def unpack_src(j, b):
        for u in range(_B // 16):
            p = sd_v[pl.ds(j * _B + u * 16, 16)]
            srcr_v[b, pl.ds(u * 16, 16)] = p & jnp.int32(0xFFFF)

    def unpack_dst(j, b):
        for u in range(_B // 16):
            p = sd_v[pl.ds(j * _B + u * 16, 16)]
            dstr_v[b, pl.ds(u * 16, 16)] = lax.shift_right_logical(
                p, jnp.int32(16))

    def start_gather(j, b):
        pltpu.async_copy(g_hbm.at[srcr_v.at[b]], rows_v.at[b], gsem[b])

    def wait_gather(j, b):
        pltpu.make_async_copy(g_hbm.at[srcr_v.at[b]], rows_v.at[b],
                              gsem[b]).wait()

    def start_scatter(j, b):
        pltpu.async_copy(rows_v.at[b], acc_s.at[dstr_v.at[b]], ssem[b],
                         add=True)

    def wait_scatter(j, b):
        pltpu.make_async_copy(rows_v.at[b], acc_s.at[dstr_v.at[b]],
                              ssem[b]).wait()

    def mul(j, b):
        @pl.loop(0, _B)
        def _(i):
            wsp = plsc.load_gather(
                w_v, [jnp.full((16,), j * _B + i, jnp.int32)])
            for c in range(_D // 16):
                sl = pl.ds(c * 16, 16)
                rows_v[b, i, sl] = rows_v[b, i, sl] * wsp

    unpack_src(0, 0)
    start_gather(0, 0)

    @pl.loop(0, _C - 1, step=_NBUF)
    def _(j0):
        for b in range(_NBUF):
            j = j0 + b
            nb = 1 - b
            unpack_src(j + 1, nb)
            if b == 0:
                @pl.when(j0 > 0)
                def _():
                    wait_scatter(j - 1, nb)
            else:
                wait_scatter(j - 1, nb)
            start_gather(j + 1, nb)
            wait_gather(j, b)
            mul(j, b)
            unpack_dst(j, b)
            start_scatter(j, b)

    _jl = _C - 1
    _bl = _jl % _NBUF
    wait_gather(_jl, _bl)
    mul(_jl, _bl)
    unpack_dst(_jl, _bl)
    wait_scatter(_jl - 1, 1 - _bl)
    pltpu.sync_copy(rows_v.at[_bl], acc_s.at[dstr_v.at[_bl]], add=True)

    plsc.subcore_barrier()

    @pl.when(sid < _NS - 1)
    def _():
        pltpu.sync_copy(acc_s.at[pl.ds(row0, _RQ)],
                        out_hbm.at[cid, pl.ds(row0, _RQ)])

    @pl.when(sid == _NS - 1)
    def _():
        pltpu.sync_copy(acc_s.at[pl.ds(row0, _RL)],
                        out_hbm.at[cid, pl.ds(row0, _RL)])


def _mm_first_block(h_ref, w_ref, o_ref):
    o_ref[...] = jnp.dot(h_ref[...], w_ref[...],
                         preferred_element_type=jnp.float32)


def _mm_mid_block(p_ref, g_ref, a_ref, b_ref, bias_ref, w_ref, o_ref):
    t = b_ref[...] * (p_ref[0] + p_ref[1] + g_ref[...]) + bias_ref[...]
    o_ref[...] = a_ref[...] * jnp.dot(t, w_ref[...],
                                      preferred_element_type=jnp.float32)


def _final_block(p_ref, g_ref, b_ref, bias_ref, o_ref):
    o_ref[...] = (b_ref[...] * (p_ref[0] + p_ref[1] + g_ref[...])
                  + bias_ref[...])


_f32 = jnp.float32
_out_nd = jax.ShapeDtypeStruct((_N, _D), _f32)


def kernel(inputs, edge_index, edge_weight, Ws, bs):
    src = edge_index[0]
    dst = edge_index[1]
    w = jnp.where(src == dst, jnp.zeros_like(edge_weight), edge_weight)
    bias = bs[:, None, :]

    sd3 = (src | (dst << 16)).reshape(_NW, _EPW)
    w3 = w.reshape(_NW, _EPW)
    zeros = jnp.zeros((_N, _D), _f32)

    deg = _sc_degrees(sd3, w3).sum(axis=0)
    a = lax.rsqrt(deg[0] + 1.0)[:, None]
    b = lax.rsqrt(deg[1] + 1.0)[:, None]

    g = a * pl.pallas_call(_mm_first_block, out_shape=_out_nd)(inputs, Ws[0])
    for l in range(3):
        parts = _sc_edge_pass(g, sd3, w3, zeros)
        if l < 2:
            g = pl.pallas_call(_mm_mid_block, out_shape=_out_nd)(
                parts, g, a, b, bias[l], Ws[l + 1])
        else:
            h = pl.pallas_call(_final_block, out_shape=_out_nd)(
                parts, g, b, bias[l])
    return h
